# Initial kernel scaffold; baseline (speedup 1.0000x reference)
import jax
import jax.numpy as jnp
from jax.experimental import pallas as pl


def kernel(x, W_enc, b_enc, W_dec):
    raise NotImplementedError("write your pallas kernel here")


# speedup vs baseline: 2.3174x; 2.3174x over previous
import jax
import jax.numpy as jnp
from jax import lax
from jax.experimental import pallas as pl
from jax.experimental.pallas import tpu as pltpu
from jax.experimental.pallas import tpu_sc as plsc

B = 128
H = 2048
L = 32768
K = 32
LT = 512
NT = L // LT
CH = 128
NCH = L // CH

_NW = 32
_RPW = B // _NW


def _enc_body(x_ref, w_ref, b_ref, z_ref):
    z = lax.dot_general(x_ref[...], w_ref[...], (((1,), (1,)), ((), ())),
                        preferred_element_type=jnp.float32)
    z_ref[...] = z + b_ref[...][None, :]


def _encode(x, W_enc, b_enc):
    return pl.pallas_call(
        _enc_body,
        grid=(NT,),
        in_specs=[
            pl.BlockSpec((B, H), lambda t: (0, 0)),
            pl.BlockSpec((LT, H), lambda t: (t, 0)),
            pl.BlockSpec((LT,), lambda t: (t,)),
        ],
        out_specs=pl.BlockSpec((B, LT), lambda t: (0, t)),
        out_shape=jax.ShapeDtypeStruct((B, L), jnp.float32),
    )(x, W_enc, b_enc)


def _sc_body(z_hbm, zsp_hbm, zrow, srow, mrow, vals_v, idx_v):
    c = lax.axis_index("c")
    s = lax.axis_index("s")
    wid = s * 2 + c
    io16 = lax.iota(jnp.int32, 16)
    zero16 = jnp.zeros((16,), jnp.float32)
    lane0 = io16 == 0

    def vmax(v):
        return plsc.cummax(v)[15]

    def vmin_i32(v):
        return -plsc.cummax(-v)[15]

    def st1(ref, i, val, dtype):
        plsc.store_scatter(ref, [jnp.full((16,), i, jnp.int32)],
                           jnp.full((16,), val, dtype), mask=lane0)

    def zero_body(i, carry):
        srow[pl.ds(i * 16, 16)] = zero16
        return carry

    lax.fori_loop(0, L // 16, zero_body, 0)

    def row_body(rr, carry):
        r = wid * _RPW + rr
        pltpu.sync_copy(z_hbm.at[r], zrow)

        def cm_body(ci, carry2):
            base = ci * CH
            m8 = zrow[pl.ds(base, 16)]
            for v in range(1, CH // 16):
                m8 = jnp.maximum(m8, zrow[pl.ds(base + v * 16, 16)])
            st1(mrow, ci, vmax(m8), jnp.float32)
            return carry2

        lax.fori_loop(0, NCH, cm_body, 0)

        def ex_body(k, carry2):
            mv = mrow[pl.ds(0, 16)]
            for v in range(1, NCH // 16):
                mv = jnp.maximum(mv, mrow[pl.ds(v * 16, 16)])
            m = vmax(mv)
            cmin = jnp.full((16,), NCH, jnp.int32)
            for v in range(NCH // 16):
                vv = mrow[pl.ds(v * 16, 16)]
                cmin = jnp.minimum(cmin, jnp.where(vv == m, io16 + v * 16, NCH))
            ci = vmin_i32(cmin)
            base = ci * CH
            pmin = jnp.full((16,), L, jnp.int32)
            for v in range(CH // 16):
                zv = zrow[pl.ds(base + v * 16, 16)]
                pmin = jnp.minimum(pmin, jnp.where(zv == m, io16 + v * 16, L))
            g = base + vmin_i32(pmin)
            st1(vals_v, k, m, jnp.float32)
            st1(idx_v, k, g, jnp.int32)
            st1(zrow, g, -jnp.inf, jnp.float32)
            m8 = zrow[pl.ds(base, 16)]
            for v in range(1, CH // 16):
                m8 = jnp.maximum(m8, zrow[pl.ds(base + v * 16, 16)])
            st1(mrow, ci, vmax(m8), jnp.float32)
            return carry2

        lax.fori_loop(0, K, ex_body, 0)

        for j in range(K // 16):
            iv = idx_v[pl.ds(j * 16, 16)]
            vv = vals_v[pl.ds(j * 16, 16)]
            plsc.store_scatter(srow, [iv], vv)
        pltpu.sync_copy(srow, zsp_hbm.at[r])
        for j in range(K // 16):
            iv = idx_v[pl.ds(j * 16, 16)]
            plsc.store_scatter(srow, [iv], zero16)
        return carry

    lax.fori_loop(0, _RPW, row_body, 0)


def _topk_scatter(z):
    mesh = plsc.VectorSubcoreMesh(core_axis_name="c", subcore_axis_name="s")
    return pl.kernel(
        _sc_body,
        out_type=jax.ShapeDtypeStruct((B, L), jnp.float32),
        mesh=mesh,
        compiler_params=pltpu.CompilerParams(needs_layout_passes=False),
        scratch_types=[
            pltpu.VMEM((L,), jnp.float32),
            pltpu.VMEM((L,), jnp.float32),
            pltpu.VMEM((NCH,), jnp.float32),
            pltpu.VMEM((K,), jnp.float32),
            pltpu.VMEM((K,), jnp.int32),
        ],
    )(z)


def _dec_body(zs_ref, w_ref, xh_ref):
    t = pl.program_id(0)
    part = lax.dot_general(zs_ref[...], w_ref[...], (((1,), (1,)), ((), ())),
                           preferred_element_type=jnp.float32)

    @pl.when(t == 0)
    def _():
        xh_ref[...] = part

    @pl.when(t != 0)
    def _():
        xh_ref[...] = xh_ref[...] + part


def _decode(z_sparse, W_dec):
    return pl.pallas_call(
        _dec_body,
        grid=(NT,),
        in_specs=[
            pl.BlockSpec((B, LT), lambda t: (0, t)),
            pl.BlockSpec((H, LT), lambda t: (0, t)),
        ],
        out_specs=pl.BlockSpec((B, H), lambda t: (0, 0)),
        out_shape=jax.ShapeDtypeStruct((B, H), jnp.float32),
    )(z_sparse, W_dec)


def kernel(x, W_enc, b_enc, W_dec):
    z = _encode(x, W_enc, b_enc)
    z_sparse = _topk_scatter(z)
    x_hat = _decode(z_sparse, W_dec)
    return (x_hat, z_sparse)
